# Initial kernel scaffold; baseline (speedup 1.0000x reference)
import jax
import jax.numpy as jnp
from jax.experimental import pallas as pl


def kernel(graph, edge_index, edge_attr, g0, mu0, sigma0, root0, bias0, g1, mu1, sigma1, root1, bias1, g2, mu2, sigma2, root2, bias2):
    raise NotImplementedError("write your pallas kernel here")


# speedup vs baseline: 1.3347x; 1.3347x over previous
import functools

import jax
import jax.numpy as jnp
from jax import lax
from jax.experimental import pallas as pl
from jax.experimental.pallas import tpu as pltpu
from jax.experimental.pallas import tpu_sc as plsc

_N = 10000
_E = 320000
_K = 10
_KH = 5
_D = 128
_DH = _KH * _D
_KP = 16
_EPS = 1e-15

_NC = 2
_NS = 16
_ET = _E // _NS
_B = 64
_NB = _ET // _B
_BT = _ET - _NB * _B
_NPT = _N // _NS

_EW = _E // (_NC * _NS)
_BC = 80


_BN = 1000


def _y_body(h_ref, gcat_ref, bias_ref, y_ref, r_ref):
    hcat = jax.lax.dot_general(
        h_ref[...], gcat_ref[...], (((1,), (0,)), ((), ())),
        preferred_element_type=jnp.float32,
        precision=jax.lax.Precision.HIGHEST)
    y_ref[0] = hcat[:, :_DH]
    y_ref[1] = hcat[:, _DH:2 * _DH]
    r_ref[...] = hcat[:, 2 * _DH:] + bias_ref[...]


def _node_transform(h, gcat, bias2d):
    grid = _N // _BN
    return pl.pallas_call(
        _y_body,
        grid=(grid,),
        in_specs=[pl.BlockSpec((_BN, _D), lambda i: (i, 0)),
                  pl.BlockSpec((_D, 2 * _DH + _D), lambda i: (0, 0)),
                  pl.BlockSpec((1, _D), lambda i: (0, 0))],
        out_specs=[pl.BlockSpec((_NC, _BN, _DH), lambda i: (0, i, 0)),
                   pl.BlockSpec((_BN, _D), lambda i: (i, 0))],
        out_shape=[jax.ShapeDtypeStruct((_NC, _N, _DH), jnp.float32),
                   jax.ShapeDtypeStruct((_N, _D), jnp.float32)],
    )(h, gcat, bias2d)


def _cnt_body(dst_hbm, zero_hbm, ones_hbm, out_hbm, dst_v, one_v, cnt_sh):
    cid = lax.axis_index("c")
    sid = lax.axis_index("s")
    wid = cid * _NS + sid

    pltpu.sync_copy(zero_hbm.at[pl.ds(sid * _NPT, _NPT)],
                    cnt_sh.at[pl.ds(sid * _NPT, _NPT)])
    pltpu.sync_copy(ones_hbm, one_v)
    plsc.subcore_barrier()

    base0 = wid * _EW

    def block_body(i, carry):
        pltpu.sync_copy(dst_hbm.at[pl.ds(base0 + i * _BC, _BC)], dst_v)
        pltpu.sync_copy(one_v, cnt_sh.at[dst_v], add=True)
        return carry
    lax.fori_loop(0, _EW // _BC, block_body, 0)

    plsc.subcore_barrier()
    pltpu.sync_copy(cnt_sh.at[pl.ds(sid * _NPT, _NPT)],
                    out_hbm.at[cid, pl.ds(sid * _NPT, _NPT)])


@functools.cache
def _get_cnt_pass():
    mesh = plsc.VectorSubcoreMesh(core_axis_name="c", subcore_axis_name="s",
                                  num_cores=_NC, num_subcores=_NS)
    return pl.kernel(
        _cnt_body,
        out_type=jax.ShapeDtypeStruct((_NC, _N, 16), jnp.float32),
        mesh=mesh,
        scratch_types=[
            pltpu.VMEM((_BC,), jnp.int32),
            pltpu.VMEM((_BC, 16), jnp.float32),
            pltpu.VMEM_SHARED((_N, 16), jnp.float32),
        ],
        compiler_params=pltpu.CompilerParams(use_tc_tiling_on_sc=False),
    )


def _edge_body(y_hbm, a0_hbm, a1_hbm, src_hbm, dst_hbm, mziv_hbm, zero_hbm,
               cnt_hbm, out_hbm, src_v, dst_v, src_t, dst_t, a0_v, a1_v,
               rows_v, msg_v, mz_v, agg_sh):
    cid = lax.axis_index("c")
    sid = lax.axis_index("s")
    base0 = sid * _ET

    pltpu.sync_copy(zero_hbm.at[pl.ds(sid * _NPT, _NPT)],
                    agg_sh.at[pl.ds(sid * _NPT, _NPT)])
    pltpu.sync_copy(mziv_hbm.at[cid], mz_v)
    plsc.subcore_barrier()

    mu0 = mz_v[0, pl.ds(0, 16)]
    mu1 = mz_v[1, pl.ds(0, 16)]
    iv0 = mz_v[2, pl.ds(0, 16)]
    iv1 = mz_v[3, pl.ds(0, 16)]

    def compute_msgs(ngroups):
        def group_one(gi, c2):
            b0 = gi * 16
            a0 = a0_v[pl.ds(b0, 16)]
            a1 = a1_v[pl.ds(b0, 16)]
            wvs = []
            for k in range(_KH):
                d0 = a0 - mu0[k]
                d1 = a1 - mu1[k]
                expo = -0.5 * (d0 * d0 * iv0[k] + d1 * d1 * iv1[k])
                wvs.append(jnp.exp(expo))
            for j in range(16):
                b = b0 + j
                accs = [wvs[0][j] * rows_v[b, pl.ds(s * 16, 16)]
                        for s in range(_D // 16)]
                for k in range(1, _KH):
                    wkj = wvs[k][j]
                    for s in range(_D // 16):
                        accs[s] = accs[s] + wkj * rows_v[b, pl.ds(k * _D + s * 16, 16)]
                for s in range(_D // 16):
                    msg_v[b, pl.ds(s * 16, 16)] = accs[s]
            return c2
        lax.fori_loop(0, ngroups, group_one, 0)

    def block_body(i, carry):
        base = base0 + i * _B
        pltpu.sync_copy(src_hbm.at[pl.ds(base, _B)], src_v)
        pltpu.sync_copy(dst_hbm.at[pl.ds(base, _B)], dst_v)
        pltpu.sync_copy(a0_hbm.at[pl.ds(base, _B)], a0_v)
        pltpu.sync_copy(a1_hbm.at[pl.ds(base, _B)], a1_v)
        pltpu.sync_copy(y_hbm.at[cid].at[src_v], rows_v)
        compute_msgs(_B // 16)
        pltpu.sync_copy(msg_v, agg_sh.at[dst_v], add=True)
        return carry
    lax.fori_loop(0, _NB, block_body, 0)

    base_t = base0 + _NB * _B
    pltpu.sync_copy(src_hbm.at[pl.ds(base_t, _BT)], src_t)
    pltpu.sync_copy(dst_hbm.at[pl.ds(base_t, _BT)], dst_t)
    pltpu.sync_copy(a0_hbm.at[pl.ds(base_t, _BT)], a0_v.at[pl.ds(0, _BT)])
    pltpu.sync_copy(a1_hbm.at[pl.ds(base_t, _BT)], a1_v.at[pl.ds(0, _BT)])
    pltpu.sync_copy(y_hbm.at[cid].at[src_t], rows_v.at[pl.ds(0, _BT)])
    compute_msgs(_BT // 16)
    pltpu.sync_copy(msg_v.at[pl.ds(0, _BT)], agg_sh.at[dst_t], add=True)

    plsc.subcore_barrier()
    pltpu.sync_copy(agg_sh.at[pl.ds(sid * _NPT, _NPT)],
                    out_hbm.at[cid, pl.ds(sid * _NPT, _NPT)])


@functools.cache
def _get_edge_pass():
    mesh = plsc.VectorSubcoreMesh(core_axis_name="c", subcore_axis_name="s",
                                  num_cores=_NC, num_subcores=_NS)
    return pl.kernel(
        _edge_body,
        out_type=jax.ShapeDtypeStruct((_NC, _N, _D), jnp.float32),
        mesh=mesh,
        scratch_types=[
            pltpu.VMEM((_B,), jnp.int32),
            pltpu.VMEM((_B,), jnp.int32),
            pltpu.VMEM((_BT,), jnp.int32),
            pltpu.VMEM((_BT,), jnp.int32),
            pltpu.VMEM((_B,), jnp.float32),
            pltpu.VMEM((_B,), jnp.float32),
            pltpu.VMEM((_B, _DH), jnp.float32),
            pltpu.VMEM((_B, _D), jnp.float32),
            pltpu.VMEM((4, 16), jnp.float32),
            pltpu.VMEM_SHARED((_N, _D), jnp.float32),
        ],
        compiler_params=pltpu.CompilerParams(use_tc_tiling_on_sc=False),
    )


def _combine_body(agg_ref, cnt_ref, r_ref, h_ref):
    s = agg_ref[0] + agg_ref[1]
    cnt = cnt_ref[0, :, 0:1] + cnt_ref[1, :, 0:1]
    o = s / jnp.maximum(cnt, 1.0) + r_ref[...]
    h_ref[...] = jnp.where(o > 0, o, jnp.exp(o) - 1.0)


def _combine(agg2, cnt2, r):
    grid = _N // _BN
    return pl.pallas_call(
        _combine_body,
        grid=(grid,),
        in_specs=[pl.BlockSpec((_NC, _BN, _D), lambda i: (0, i, 0)),
                  pl.BlockSpec((_NC, _BN, 16), lambda i: (0, i, 0)),
                  pl.BlockSpec((_BN, _D), lambda i: (i, 0))],
        out_specs=pl.BlockSpec((_BN, _D), lambda i: (i, 0)),
        out_shape=jax.ShapeDtypeStruct((_N, _D), jnp.float32),
    )(agg2, cnt2, r)


def kernel(graph, edge_index, edge_attr,
           g0, mu0, sigma0, root0, bias0,
           g1, mu1, sigma1, root1, bias1,
           g2, mu2, sigma2, root2, bias2):
    src = edge_index[0]
    dst = edge_index[1]
    ea0 = jnp.asarray(edge_attr[:, 0])
    ea1 = jnp.asarray(edge_attr[:, 1])
    zero = jnp.zeros((_N, _D), jnp.float32)
    zero16 = jnp.zeros((_N, 16), jnp.float32)

    mzivs, gcats, biases = [], [], []
    for (g, mu, sigma, root, bias) in ((g0, mu0, sigma0, root0, bias0),
                                       (g1, mu1, sigma1, root1, bias1),
                                       (g2, mu2, sigma2, root2, bias2)):
        mu_t = mu.T
        iv_t = 1.0 / (_EPS + sigma.T ** 2)
        packed = jnp.concatenate([mu_t, iv_t], axis=0)
        mzivs.append(jnp.stack([
            jnp.pad(packed[:, c * _KH:(c + 1) * _KH], ((0, 0), (0, _KP - _KH)))
            for c in range(_NC)]))
        gcats.append(jnp.concatenate([g, root], axis=1))
        biases.append(bias.reshape(1, _D))

    ones16 = jnp.ones((_BC, 16), jnp.float32)
    cnt2 = _get_cnt_pass()(dst, zero16, ones16)

    outs = []
    h = graph
    for l in range(3):
        y, r = _node_transform(h, gcats[l], biases[l])
        agg2 = _get_edge_pass()(y, ea0, ea1, src, dst, mzivs[l], zero, cnt2)
        h = _combine(agg2, cnt2, r)
        outs.append(h)
    return tuple(outs)


# speedup vs baseline: 1.4739x; 1.1043x over previous
import functools

import jax
import jax.numpy as jnp
from jax import lax
from jax.experimental import pallas as pl
from jax.experimental.pallas import tpu as pltpu
from jax.experimental.pallas import tpu_sc as plsc

_N = 10000
_E = 320000
_K = 10
_KH = 5
_D = 128
_DH = _KH * _D
_KP = 16
_EPS = 1e-15

_NC = 2
_NS = 16
_ET = _E // _NS
_B = 32
_SB = 5
_NSB = _ET // (_B * _SB)
_ER = _E // _B
_RPT = _ET // _B
_NPT = _N // _NS

_EW = _E // (_NC * _NS)
_BC = 80


_BN = 1000


def _y_body(h_ref, gcat_ref, bias_ref, y_ref, r_ref):
    hcat = jax.lax.dot_general(
        h_ref[...], gcat_ref[...], (((1,), (0,)), ((), ())),
        preferred_element_type=jnp.float32,
        precision=jax.lax.Precision.HIGHEST)
    y_ref[0] = hcat[:, :_DH]
    y_ref[1] = hcat[:, _DH:2 * _DH]
    r_ref[...] = hcat[:, 2 * _DH:] + bias_ref[...]


def _node_transform(h, gcat, bias2d):
    grid = _N // _BN
    return pl.pallas_call(
        _y_body,
        grid=(grid,),
        in_specs=[pl.BlockSpec((_BN, _D), lambda i: (i, 0)),
                  pl.BlockSpec((_D, 2 * _DH + _D), lambda i: (0, 0)),
                  pl.BlockSpec((1, _D), lambda i: (0, 0))],
        out_specs=[pl.BlockSpec((_NC, _BN, _DH), lambda i: (0, i, 0)),
                   pl.BlockSpec((_BN, _D), lambda i: (i, 0))],
        out_shape=[jax.ShapeDtypeStruct((_NC, _N, _DH), jnp.float32),
                   jax.ShapeDtypeStruct((_N, _D), jnp.float32)],
    )(h, gcat, bias2d)


def _cnt_body(dst_hbm, zero_hbm, ones_hbm, out_hbm, dst_v, one_v, cnt_sh):
    cid = lax.axis_index("c")
    sid = lax.axis_index("s")
    wid = cid * _NS + sid

    pltpu.sync_copy(zero_hbm.at[pl.ds(sid * _NPT, _NPT)],
                    cnt_sh.at[pl.ds(sid * _NPT, _NPT)])
    pltpu.sync_copy(ones_hbm, one_v)
    plsc.subcore_barrier()

    base0 = wid * _EW

    def block_body(i, carry):
        pltpu.sync_copy(dst_hbm.at[pl.ds(base0 + i * _BC, _BC)], dst_v)
        pltpu.sync_copy(one_v, cnt_sh.at[dst_v], add=True)
        return carry
    lax.fori_loop(0, _EW // _BC, block_body, 0)

    plsc.subcore_barrier()
    pltpu.sync_copy(cnt_sh.at[pl.ds(sid * _NPT, _NPT)],
                    out_hbm.at[cid, pl.ds(sid * _NPT, _NPT)])


@functools.cache
def _get_cnt_pass():
    mesh = plsc.VectorSubcoreMesh(core_axis_name="c", subcore_axis_name="s",
                                  num_cores=_NC, num_subcores=_NS)
    return pl.kernel(
        _cnt_body,
        out_type=jax.ShapeDtypeStruct((_NC, _N, 16), jnp.float32),
        mesh=mesh,
        scratch_types=[
            pltpu.VMEM((_BC,), jnp.int32),
            pltpu.VMEM((_BC, 16), jnp.float32),
            pltpu.VMEM_SHARED((_N, 16), jnp.float32),
        ],
        compiler_params=pltpu.CompilerParams(use_tc_tiling_on_sc=False),
    )


def _edge_body(y_hbm, a0_hbm, a1_hbm, src_hbm, dst_hbm, mziv_hbm, zero_hbm,
               cnt_hbm, out_hbm, src_v, dst_v, a0_v, a1_v,
               rows0_v, rows1_v, msg_v, mz_v, agg_sh, gsem0, gsem1):
    cid = lax.axis_index("c")
    sid = lax.axis_index("s")
    row0 = sid * _RPT

    pltpu.sync_copy(zero_hbm.at[pl.ds(sid * _NPT, _NPT)],
                    agg_sh.at[pl.ds(sid * _NPT, _NPT)])
    pltpu.sync_copy(mziv_hbm.at[cid], mz_v)
    plsc.subcore_barrier()

    mu0 = mz_v[0, pl.ds(0, 16)]
    mu1 = mz_v[1, pl.ds(0, 16)]
    iv0 = mz_v[2, pl.ds(0, 16)]
    iv1 = mz_v[3, pl.ds(0, 16)]

    rows_bufs = (rows0_v, rows1_v)
    gsems = (gsem0, gsem1)

    def gather_start(j, slot):
        pltpu.async_copy(y_hbm.at[cid].at[src_v.at[j]], rows_bufs[slot],
                         gsems[slot])

    def gather_wait(j, slot):
        pltpu.make_async_copy(y_hbm.at[cid].at[src_v.at[j]], rows_bufs[slot],
                              gsems[slot]).wait()

    def compute_block(j, slot):
        rows_v = rows_bufs[slot]

        def group_one(gi, c2):
            b0 = gi * 16
            a0 = a0_v[j, pl.ds(b0, 16)]
            a1 = a1_v[j, pl.ds(b0, 16)]
            wvs = []
            for k in range(_KH):
                d0 = a0 - mu0[k]
                d1 = a1 - mu1[k]
                expo = -0.5 * (d0 * d0 * iv0[k] + d1 * d1 * iv1[k])
                wvs.append(jnp.exp(expo))
            for jj in range(16):
                b = b0 + jj
                accs = [wvs[0][jj] * rows_v[b, pl.ds(s * 16, 16)]
                        for s in range(_D // 16)]
                for k in range(1, _KH):
                    wkj = wvs[k][jj]
                    for s in range(_D // 16):
                        accs[s] = accs[s] + wkj * rows_v[b, pl.ds(k * _D + s * 16, 16)]
                for s in range(_D // 16):
                    msg_v[b, pl.ds(s * 16, 16)] = accs[s]
            return c2
        lax.fori_loop(0, _B // 16, group_one, 0)
        pltpu.sync_copy(msg_v, agg_sh.at[dst_v.at[j]], add=True)

    def super_body(g, carry):
        r = row0 + g * _SB
        pltpu.sync_copy(src_hbm.at[pl.ds(r, _SB)], src_v)
        pltpu.sync_copy(dst_hbm.at[pl.ds(r, _SB)], dst_v)
        pltpu.sync_copy(a0_hbm.at[pl.ds(r, _SB)], a0_v)
        pltpu.sync_copy(a1_hbm.at[pl.ds(r, _SB)], a1_v)
        gather_start(0, 0)
        gather_start(1, 1)
        for j in range(_SB):
            gather_wait(j, j % 2)
            if j + 2 < _SB:
                pass
            compute_block(j, j % 2)
            if j + 2 < _SB:
                gather_start(j + 2, j % 2)
        return carry

    lax.fori_loop(0, _NSB, super_body, 0)

    plsc.subcore_barrier()
    pltpu.sync_copy(agg_sh.at[pl.ds(sid * _NPT, _NPT)],
                    out_hbm.at[cid, pl.ds(sid * _NPT, _NPT)])


@functools.cache
def _get_edge_pass():
    mesh = plsc.VectorSubcoreMesh(core_axis_name="c", subcore_axis_name="s",
                                  num_cores=_NC, num_subcores=_NS)
    return pl.kernel(
        _edge_body,
        out_type=jax.ShapeDtypeStruct((_NC, _N, _D), jnp.float32),
        mesh=mesh,
        scratch_types=[
            pltpu.VMEM((_SB, _B), jnp.int32),
            pltpu.VMEM((_SB, _B), jnp.int32),
            pltpu.VMEM((_SB, _B), jnp.float32),
            pltpu.VMEM((_SB, _B), jnp.float32),
            pltpu.VMEM((_B, _DH), jnp.float32),
            pltpu.VMEM((_B, _DH), jnp.float32),
            pltpu.VMEM((_B, _D), jnp.float32),
            pltpu.VMEM((4, 16), jnp.float32),
            pltpu.VMEM_SHARED((_N, _D), jnp.float32),
            pltpu.SemaphoreType.DMA,
            pltpu.SemaphoreType.DMA,
        ],
        compiler_params=pltpu.CompilerParams(use_tc_tiling_on_sc=False),
    )


def _combine_body(agg_ref, cnt_ref, r_ref, h_ref):
    s = agg_ref[0] + agg_ref[1]
    cnt = cnt_ref[0, :, 0:1] + cnt_ref[1, :, 0:1]
    o = s / jnp.maximum(cnt, 1.0) + r_ref[...]
    h_ref[...] = jnp.where(o > 0, o, jnp.exp(o) - 1.0)


def _combine(agg2, cnt2, r):
    grid = _N // _BN
    return pl.pallas_call(
        _combine_body,
        grid=(grid,),
        in_specs=[pl.BlockSpec((_NC, _BN, _D), lambda i: (0, i, 0)),
                  pl.BlockSpec((_NC, _BN, 16), lambda i: (0, i, 0)),
                  pl.BlockSpec((_BN, _D), lambda i: (i, 0))],
        out_specs=pl.BlockSpec((_BN, _D), lambda i: (i, 0)),
        out_shape=jax.ShapeDtypeStruct((_N, _D), jnp.float32),
    )(agg2, cnt2, r)


def kernel(graph, edge_index, edge_attr,
           g0, mu0, sigma0, root0, bias0,
           g1, mu1, sigma1, root1, bias1,
           g2, mu2, sigma2, root2, bias2):
    src = edge_index[0]
    dst = edge_index[1]
    src2 = src.reshape(_ER, _B)
    dst2 = dst.reshape(_ER, _B)
    ea02 = jnp.asarray(edge_attr[:, 0]).reshape(_ER, _B)
    ea12 = jnp.asarray(edge_attr[:, 1]).reshape(_ER, _B)
    zero = jnp.zeros((_N, _D), jnp.float32)
    zero16 = jnp.zeros((_N, 16), jnp.float32)

    mzivs, gcats, biases = [], [], []
    for (g, mu, sigma, root, bias) in ((g0, mu0, sigma0, root0, bias0),
                                       (g1, mu1, sigma1, root1, bias1),
                                       (g2, mu2, sigma2, root2, bias2)):
        mu_t = mu.T
        iv_t = 1.0 / (_EPS + sigma.T ** 2)
        packed = jnp.concatenate([mu_t, iv_t], axis=0)
        mzivs.append(jnp.stack([
            jnp.pad(packed[:, c * _KH:(c + 1) * _KH], ((0, 0), (0, _KP - _KH)))
            for c in range(_NC)]))
        gcats.append(jnp.concatenate([g, root], axis=1))
        biases.append(bias.reshape(1, _D))

    ones16 = jnp.ones((_BC, 16), jnp.float32)
    cnt2 = _get_cnt_pass()(dst, zero16, ones16)

    outs = []
    h = graph
    for l in range(3):
        y, r = _node_transform(h, gcats[l], biases[l])
        agg2 = _get_edge_pass()(y, ea02, ea12, src2, dst2, mzivs[l], zero, cnt2)
        h = _combine(agg2, cnt2, r)
        outs.append(h)
    return tuple(outs)


# speedup vs baseline: 1.5020x; 1.0191x over previous
import functools

import jax
import jax.numpy as jnp
import numpy as np
from jax import lax
from jax.experimental import pallas as pl
from jax.experimental.pallas import tpu as pltpu
from jax.experimental.pallas import tpu_sc as plsc

_N = 10000
_E = 320000
_K = 10
_KH = 5
_D = 128
_DH = _KH * _D
_KP = 16
_EPS = 1e-15

_NC = 2
_NS = 16
_ET = _E // _NS
_B = 32
_SB = 5
_NSLOT = 4
_NSB = _ET // (_B * _SB)
_ER = _E // _B
_RPT = _ET // _B
_NPT = _N // _NS

_EW = _E // (_NC * _NS)
_BC = 80
_XPROF_GATHER_ONLY = False


def _half_perm(c):
    perm = []
    for g0 in range(20):
        k, s2 = divmod(g0, 4)
        base = (c * _KH + k) * _D + s2 * 32
        for i in range(16):
            perm.append(base + i)
            perm.append(base + 16 + i)
    return np.asarray(perm, dtype=np.int32)


_PERMS = (_half_perm(0), _half_perm(1))


_BN = 1000


def _y_body(h_ref, gcat_ref, bias_ref, y_ref, r_ref):
    hcat = jax.lax.dot_general(
        h_ref[...], gcat_ref[...], (((1,), (0,)), ((), ())),
        preferred_element_type=jnp.float32,
        precision=jax.lax.Precision.HIGHEST)
    y_ref[0] = hcat[:, :_DH].astype(jnp.bfloat16)
    y_ref[1] = hcat[:, _DH:2 * _DH].astype(jnp.bfloat16)
    r_ref[...] = hcat[:, 2 * _DH:] + bias_ref[...]


def _node_transform(h, gcat, bias2d):
    grid = _N // _BN
    return pl.pallas_call(
        _y_body,
        grid=(grid,),
        in_specs=[pl.BlockSpec((_BN, _D), lambda i: (i, 0)),
                  pl.BlockSpec((_D, 2 * _DH + _D), lambda i: (0, 0)),
                  pl.BlockSpec((1, _D), lambda i: (0, 0))],
        out_specs=[pl.BlockSpec((_NC, _BN, _DH), lambda i: (0, i, 0)),
                   pl.BlockSpec((_BN, _D), lambda i: (i, 0))],
        out_shape=[jax.ShapeDtypeStruct((_NC, _N, _DH), jnp.bfloat16),
                   jax.ShapeDtypeStruct((_N, _D), jnp.float32)],
    )(h, gcat, bias2d)


def _cnt_body(dst_hbm, zero_hbm, ones_hbm, out_hbm, dst_v, one_v, cnt_sh):
    cid = lax.axis_index("c")
    sid = lax.axis_index("s")
    wid = cid * _NS + sid

    pltpu.sync_copy(zero_hbm.at[pl.ds(sid * _NPT, _NPT)],
                    cnt_sh.at[pl.ds(sid * _NPT, _NPT)])
    pltpu.sync_copy(ones_hbm, one_v)
    plsc.subcore_barrier()

    base0 = wid * _EW

    def block_body(i, carry):
        pltpu.sync_copy(dst_hbm.at[pl.ds(base0 + i * _BC, _BC)], dst_v)
        pltpu.sync_copy(one_v, cnt_sh.at[dst_v], add=True)
        return carry
    lax.fori_loop(0, _EW // _BC, block_body, 0)

    plsc.subcore_barrier()
    pltpu.sync_copy(cnt_sh.at[pl.ds(sid * _NPT, _NPT)],
                    out_hbm.at[cid, pl.ds(sid * _NPT, _NPT)])


@functools.cache
def _get_cnt_pass():
    mesh = plsc.VectorSubcoreMesh(core_axis_name="c", subcore_axis_name="s",
                                  num_cores=_NC, num_subcores=_NS)
    return pl.kernel(
        _cnt_body,
        out_type=jax.ShapeDtypeStruct((_NC, _N, 16), jnp.float32),
        mesh=mesh,
        scratch_types=[
            pltpu.VMEM((_BC,), jnp.int32),
            pltpu.VMEM((_BC, 16), jnp.float32),
            pltpu.VMEM_SHARED((_N, 16), jnp.float32),
        ],
        compiler_params=pltpu.CompilerParams(use_tc_tiling_on_sc=False),
    )


def _edge_body(y_hbm, a0_hbm, a1_hbm, src_hbm, dst_hbm, mziv_hbm, zero_hbm,
               cnt_hbm, out_hbm, src_v, dst_v, a0_v, a1_v,
               rows0_v, rows1_v, rows2_v, rows3_v, msg_v, mz_v, agg_sh,
               gsem0, gsem1, gsem2, gsem3):
    cid = lax.axis_index("c")
    sid = lax.axis_index("s")
    row0 = sid * _RPT

    pltpu.sync_copy(zero_hbm.at[pl.ds(sid * _NPT, _NPT)],
                    agg_sh.at[pl.ds(sid * _NPT, _NPT)])
    pltpu.sync_copy(mziv_hbm.at[cid], mz_v)
    plsc.subcore_barrier()

    mu0 = mz_v[0, pl.ds(0, 16)]
    mu1 = mz_v[1, pl.ds(0, 16)]
    iv0 = mz_v[2, pl.ds(0, 16)]
    iv1 = mz_v[3, pl.ds(0, 16)]

    rows_bufs = (rows0_v, rows1_v, rows2_v, rows3_v)
    gsems = (gsem0, gsem1, gsem2, gsem3)

    def gather_start(j, slot):
        pltpu.async_copy(y_hbm.at[cid].at[src_v.at[j]], rows_bufs[slot],
                         gsems[slot])

    def gather_wait(j, slot):
        pltpu.make_async_copy(y_hbm.at[cid].at[src_v.at[j]], rows_bufs[slot],
                              gsems[slot]).wait()

    def compute_block(j, slot):
        rows_v = rows_bufs[slot]

        def group_one(gi, c2):
            b0 = gi * 16
            a0 = a0_v[j, pl.ds(b0, 16)]
            a1 = a1_v[j, pl.ds(b0, 16)]
            wvs = []
            for k in range(_KH):
                d0 = a0 - mu0[k]
                d1 = a1 - mu1[k]
                expo = -0.5 * (d0 * d0 * iv0[k] + d1 * d1 * iv1[k])
                wvs.append(jnp.exp(expo))
            for jj in range(16):
                b = b0 + jj
                accs = [None] * (_D // 16)
                for k in range(_KH):
                    wkj = wvs[k][jj]
                    for s2 in range(_D // 32):
                        v = rows_v[b, pl.ds(k * _D + s2 * 32, 32)]
                        lo, hi = plsc.unpack(
                            v, format=plsc.PackFormat.INTERLEAVED)
                        for s, part in ((2 * s2, lo), (2 * s2 + 1, hi)):
                            t = wkj * part
                            accs[s] = t if accs[s] is None else accs[s] + t
                for s in range(_D // 16):
                    msg_v[b, pl.ds(s * 16, 16)] = accs[s]
            return c2
        if not _XPROF_GATHER_ONLY:
            lax.fori_loop(0, _B // 16, group_one, 0)
            pltpu.sync_copy(msg_v, agg_sh.at[dst_v.at[j]], add=True)

    def super_body(g, carry):
        r = row0 + g * _SB
        pltpu.sync_copy(src_hbm.at[pl.ds(r, _SB)], src_v)
        pltpu.sync_copy(dst_hbm.at[pl.ds(r, _SB)], dst_v)
        pltpu.sync_copy(a0_hbm.at[pl.ds(r, _SB)], a0_v)
        pltpu.sync_copy(a1_hbm.at[pl.ds(r, _SB)], a1_v)
        for j in range(_NSLOT - 1):
            gather_start(j, j % _NSLOT)
        for j in range(_SB):
            gather_wait(j, j % _NSLOT)
            compute_block(j, j % _NSLOT)
            if j + _NSLOT - 1 < _SB:
                gather_start(j + _NSLOT - 1, (j + _NSLOT - 1) % _NSLOT)
        return carry

    lax.fori_loop(0, _NSB, super_body, 0)

    plsc.subcore_barrier()
    pltpu.sync_copy(agg_sh.at[pl.ds(sid * _NPT, _NPT)],
                    out_hbm.at[cid, pl.ds(sid * _NPT, _NPT)])


@functools.cache
def _get_edge_pass():
    mesh = plsc.VectorSubcoreMesh(core_axis_name="c", subcore_axis_name="s",
                                  num_cores=_NC, num_subcores=_NS)
    return pl.kernel(
        _edge_body,
        out_type=jax.ShapeDtypeStruct((_NC, _N, _D), jnp.float32),
        mesh=mesh,
        scratch_types=[
            pltpu.VMEM((_SB, _B), jnp.int32),
            pltpu.VMEM((_SB, _B), jnp.int32),
            pltpu.VMEM((_SB, _B), jnp.float32),
            pltpu.VMEM((_SB, _B), jnp.float32),
            pltpu.VMEM((_B, _DH), jnp.bfloat16),
            pltpu.VMEM((_B, _DH), jnp.bfloat16),
            pltpu.VMEM((_B, _DH), jnp.bfloat16),
            pltpu.VMEM((_B, _DH), jnp.bfloat16),
            pltpu.VMEM((_B, _D), jnp.float32),
            pltpu.VMEM((4, 16), jnp.float32),
            pltpu.VMEM_SHARED((_N, _D), jnp.float32),
            pltpu.SemaphoreType.DMA,
            pltpu.SemaphoreType.DMA,
            pltpu.SemaphoreType.DMA,
            pltpu.SemaphoreType.DMA,
        ],
        compiler_params=pltpu.CompilerParams(use_tc_tiling_on_sc=False,
                                             needs_layout_passes=False),
    )


def _combine_body(agg_ref, cnt_ref, r_ref, h_ref):
    s = agg_ref[0] + agg_ref[1]
    cnt = cnt_ref[0, :, 0:1] + cnt_ref[1, :, 0:1]
    o = s / jnp.maximum(cnt, 1.0) + r_ref[...]
    h_ref[...] = jnp.where(o > 0, o, jnp.exp(o) - 1.0)


def _combine(agg2, cnt2, r):
    grid = _N // _BN
    return pl.pallas_call(
        _combine_body,
        grid=(grid,),
        in_specs=[pl.BlockSpec((_NC, _BN, _D), lambda i: (0, i, 0)),
                  pl.BlockSpec((_NC, _BN, 16), lambda i: (0, i, 0)),
                  pl.BlockSpec((_BN, _D), lambda i: (i, 0))],
        out_specs=pl.BlockSpec((_BN, _D), lambda i: (i, 0)),
        out_shape=jax.ShapeDtypeStruct((_N, _D), jnp.float32),
    )(agg2, cnt2, r)


def kernel(graph, edge_index, edge_attr,
           g0, mu0, sigma0, root0, bias0,
           g1, mu1, sigma1, root1, bias1,
           g2, mu2, sigma2, root2, bias2):
    src = edge_index[0]
    dst = edge_index[1]
    src2 = src.reshape(_ER, _B)
    dst2 = dst.reshape(_ER, _B)
    ea02 = jnp.asarray(edge_attr[:, 0]).reshape(_ER, _B)
    ea12 = jnp.asarray(edge_attr[:, 1]).reshape(_ER, _B)
    zero = jnp.zeros((_N, _D), jnp.float32)
    zero16 = jnp.zeros((_N, 16), jnp.float32)

    mzivs, gcats, biases = [], [], []
    for (g, mu, sigma, root, bias) in ((g0, mu0, sigma0, root0, bias0),
                                       (g1, mu1, sigma1, root1, bias1),
                                       (g2, mu2, sigma2, root2, bias2)):
        mu_t = mu.T
        iv_t = 1.0 / (_EPS + sigma.T ** 2)
        packed = jnp.concatenate([mu_t, iv_t], axis=0)
        mzivs.append(jnp.stack([
            jnp.pad(packed[:, c * _KH:(c + 1) * _KH], ((0, 0), (0, _KP - _KH)))
            for c in range(_NC)]))
        gcats.append(jnp.concatenate(
            [g[:, _PERMS[0]], g[:, _PERMS[1]], root], axis=1))
        biases.append(bias.reshape(1, _D))

    ones16 = jnp.ones((_BC, 16), jnp.float32)
    cnt2 = _get_cnt_pass()(dst, zero16, ones16)

    outs = []
    h = graph
    for l in range(3):
        y, r = _node_transform(h, gcats[l], biases[l])
        agg2 = _get_edge_pass()(y, ea02, ea12, src2, dst2, mzivs[l], zero, cnt2)
        h = _combine(agg2, cnt2, r)
        outs.append(h)
    return tuple(outs)


# speedup vs baseline: 1.9988x; 1.3308x over previous
import functools

import jax
import jax.numpy as jnp
import numpy as np
from jax import lax
from jax.experimental import pallas as pl
from jax.experimental.pallas import tpu as pltpu
from jax.experimental.pallas import tpu_sc as plsc

_N = 10000
_E = 320000
_K = 10
_KH = 5
_D = 128
_DH = _KH * _D
_KP = 16
_EPS = 1e-15

_NC = 2
_NS = 16
_ET = _E // _NS
_B = 32
_SB = 5
_NSLOT = 4
_NSB = _ET // (_B * _SB)
_ER = _E // _B
_RPT = _ET // _B
_NPT = _N // _NS

_EW = _E // (_NC * _NS)
_BC = 80
_XPROF_GATHER_ONLY = False
_XPROF_NO_SCATTER = False


def _half_perm(c):
    perm = []
    for g0 in range(20):
        k, s2 = divmod(g0, 4)
        base = (c * _KH + k) * _D + s2 * 32
        for i in range(16):
            perm.append(base + i)
            perm.append(base + 16 + i)
    return np.asarray(perm, dtype=np.int32)


_PERMS = (_half_perm(0), _half_perm(1))


_BN = 1000


def _y_body(h_ref, gcat_ref, bias_ref, y_ref, r_ref):
    hcat = jax.lax.dot_general(
        h_ref[...], gcat_ref[...], (((1,), (0,)), ((), ())),
        preferred_element_type=jnp.float32,
        precision=jax.lax.Precision.HIGHEST)
    y_ref[0] = hcat[:, :_DH].astype(jnp.bfloat16)
    y_ref[1] = hcat[:, _DH:2 * _DH].astype(jnp.bfloat16)
    r_ref[...] = hcat[:, 2 * _DH:] + bias_ref[...]


def _node_transform(h, gcat, bias2d):
    grid = _N // _BN
    return pl.pallas_call(
        _y_body,
        grid=(grid,),
        in_specs=[pl.BlockSpec((_BN, _D), lambda i: (i, 0)),
                  pl.BlockSpec((_D, 2 * _DH + _D), lambda i: (0, 0)),
                  pl.BlockSpec((1, _D), lambda i: (0, 0))],
        out_specs=[pl.BlockSpec((_NC, _BN, _DH), lambda i: (0, i, 0)),
                   pl.BlockSpec((_BN, _D), lambda i: (i, 0))],
        out_shape=[jax.ShapeDtypeStruct((_NC, _N, _DH), jnp.bfloat16),
                   jax.ShapeDtypeStruct((_N, _D), jnp.float32)],
    )(h, gcat, bias2d)


def _cnt_body(dst_hbm, zero_hbm, ones_hbm, out_hbm, dst_v, one_v, cnt_sh):
    cid = lax.axis_index("c")
    sid = lax.axis_index("s")
    wid = cid * _NS + sid

    pltpu.sync_copy(zero_hbm.at[pl.ds(sid * _NPT, _NPT)],
                    cnt_sh.at[pl.ds(sid * _NPT, _NPT)])
    pltpu.sync_copy(ones_hbm, one_v)
    plsc.subcore_barrier()

    base0 = wid * _EW

    def block_body(i, carry):
        pltpu.sync_copy(dst_hbm.at[pl.ds(base0 + i * _BC, _BC)], dst_v)
        pltpu.sync_copy(one_v, cnt_sh.at[dst_v], add=True)
        return carry
    lax.fori_loop(0, _EW // _BC, block_body, 0)

    plsc.subcore_barrier()
    pltpu.sync_copy(cnt_sh.at[pl.ds(sid * _NPT, _NPT)],
                    out_hbm.at[cid, pl.ds(sid * _NPT, _NPT)])


@functools.cache
def _get_cnt_pass():
    mesh = plsc.VectorSubcoreMesh(core_axis_name="c", subcore_axis_name="s",
                                  num_cores=_NC, num_subcores=_NS)
    return pl.kernel(
        _cnt_body,
        out_type=jax.ShapeDtypeStruct((_NC, _N, 16), jnp.float32),
        mesh=mesh,
        scratch_types=[
            pltpu.VMEM((_BC,), jnp.int32),
            pltpu.VMEM((_BC, 16), jnp.float32),
            pltpu.VMEM_SHARED((_N, 16), jnp.float32),
        ],
        compiler_params=pltpu.CompilerParams(use_tc_tiling_on_sc=False),
    )


def _edge_body(y_hbm, a0_hbm, a1_hbm, src_hbm, dst_hbm, mziv_hbm, zero_hbm,
               cnt_hbm, out_hbm, src_v, dst_v, a0_v, a1_v,
               rows0_v, rows1_v, rows2_v, rows3_v, msg_v, mz_v, agg_sh,
               gsem0, gsem1, gsem2, gsem3):
    cid = lax.axis_index("c")
    sid = lax.axis_index("s")
    row0 = sid * _RPT

    pltpu.sync_copy(zero_hbm.at[pl.ds(sid * _NPT, _NPT)],
                    agg_sh.at[pl.ds(sid * _NPT, _NPT)])
    pltpu.sync_copy(mziv_hbm.at[cid], mz_v)
    plsc.subcore_barrier()

    mu0 = mz_v[0, pl.ds(0, 16)]
    mu1 = mz_v[1, pl.ds(0, 16)]
    iv0 = mz_v[2, pl.ds(0, 16)]
    iv1 = mz_v[3, pl.ds(0, 16)]

    rows_bufs = (rows0_v, rows1_v, rows2_v, rows3_v)
    gsems = (gsem0, gsem1, gsem2, gsem3)

    def gather_start(j, slot):
        pltpu.async_copy(y_hbm.at[cid].at[src_v.at[j]], rows_bufs[slot],
                         gsems[slot])

    def gather_wait(j, slot):
        pltpu.make_async_copy(y_hbm.at[cid].at[src_v.at[j]], rows_bufs[slot],
                              gsems[slot]).wait()

    def compute_block(j, slot):
        rows_v = rows_bufs[slot]

        def group_one(gi, c2):
            b0 = gi * 16
            a0 = a0_v[j, pl.ds(b0, 16)]
            a1 = a1_v[j, pl.ds(b0, 16)]
            wvs = []
            for k in range(_KH):
                d0 = a0 - mu0[k]
                d1 = a1 - mu1[k]
                expo = -0.5 * (d0 * d0 * iv0[k] + d1 * d1 * iv1[k])
                wvs.append(jnp.exp(expo))
            for jj in range(16):
                b = b0 + jj
                accs = [None] * (_D // 32)
                for k in range(_KH):
                    wsp = lax.gather(
                        wvs[k], jnp.full((16, 1), jj, jnp.int32),
                        lax.GatherDimensionNumbers(
                            offset_dims=(), collapsed_slice_dims=(0,),
                            start_index_map=(0,)),
                        (1,), mode=lax.GatherScatterMode.PROMISE_IN_BOUNDS)
                    wkj = plsc.pack(wsp, wsp,
                                    format=plsc.PackFormat.INTERLEAVED)
                    for s2 in range(_D // 32):
                        t = wkj * rows_v[b, pl.ds(k * _D + s2 * 32, 32)]
                        accs[s2] = t if accs[s2] is None else accs[s2] + t
                for s2 in range(_D // 32):
                    lo, hi = plsc.unpack(
                        accs[s2], format=plsc.PackFormat.INTERLEAVED)
                    msg_v[b, pl.ds(2 * s2 * 16, 16)] = lo
                    msg_v[b, pl.ds((2 * s2 + 1) * 16, 16)] = hi
            return c2
        if not _XPROF_GATHER_ONLY:
            lax.fori_loop(0, _B // 16, group_one, 0)
            if not _XPROF_NO_SCATTER:
                pltpu.sync_copy(msg_v, agg_sh.at[dst_v.at[j]], add=True)

    def super_body(g, carry):
        r = row0 + g * _SB
        pltpu.sync_copy(src_hbm.at[pl.ds(r, _SB)], src_v)
        pltpu.sync_copy(dst_hbm.at[pl.ds(r, _SB)], dst_v)
        pltpu.sync_copy(a0_hbm.at[pl.ds(r, _SB)], a0_v)
        pltpu.sync_copy(a1_hbm.at[pl.ds(r, _SB)], a1_v)
        for j in range(_NSLOT - 1):
            gather_start(j, j % _NSLOT)
        for j in range(_SB):
            gather_wait(j, j % _NSLOT)
            compute_block(j, j % _NSLOT)
            if j + _NSLOT - 1 < _SB:
                gather_start(j + _NSLOT - 1, (j + _NSLOT - 1) % _NSLOT)
        return carry

    lax.fori_loop(0, _NSB, super_body, 0)

    plsc.subcore_barrier()
    pltpu.sync_copy(agg_sh.at[pl.ds(sid * _NPT, _NPT)],
                    out_hbm.at[cid, pl.ds(sid * _NPT, _NPT)])


@functools.cache
def _get_edge_pass():
    mesh = plsc.VectorSubcoreMesh(core_axis_name="c", subcore_axis_name="s",
                                  num_cores=_NC, num_subcores=_NS)
    return pl.kernel(
        _edge_body,
        out_type=jax.ShapeDtypeStruct((_NC, _N, _D), jnp.float32),
        mesh=mesh,
        scratch_types=[
            pltpu.VMEM((_SB, _B), jnp.int32),
            pltpu.VMEM((_SB, _B), jnp.int32),
            pltpu.VMEM((_SB, _B), jnp.float32),
            pltpu.VMEM((_SB, _B), jnp.float32),
            pltpu.VMEM((_B, _DH), jnp.bfloat16),
            pltpu.VMEM((_B, _DH), jnp.bfloat16),
            pltpu.VMEM((_B, _DH), jnp.bfloat16),
            pltpu.VMEM((_B, _DH), jnp.bfloat16),
            pltpu.VMEM((_B, _D), jnp.float32),
            pltpu.VMEM((4, 16), jnp.float32),
            pltpu.VMEM_SHARED((_N, _D), jnp.float32),
            pltpu.SemaphoreType.DMA,
            pltpu.SemaphoreType.DMA,
            pltpu.SemaphoreType.DMA,
            pltpu.SemaphoreType.DMA,
        ],
        compiler_params=pltpu.CompilerParams(use_tc_tiling_on_sc=False,
                                             needs_layout_passes=False),
    )


def _combine_body(agg_ref, cnt_ref, r_ref, h_ref):
    s = agg_ref[0] + agg_ref[1]
    cnt = cnt_ref[0, :, 0:1] + cnt_ref[1, :, 0:1]
    o = s / jnp.maximum(cnt, 1.0) + r_ref[...]
    h_ref[...] = jnp.where(o > 0, o, jnp.exp(o) - 1.0)


def _combine(agg2, cnt2, r):
    grid = _N // _BN
    return pl.pallas_call(
        _combine_body,
        grid=(grid,),
        in_specs=[pl.BlockSpec((_NC, _BN, _D), lambda i: (0, i, 0)),
                  pl.BlockSpec((_NC, _BN, 16), lambda i: (0, i, 0)),
                  pl.BlockSpec((_BN, _D), lambda i: (i, 0))],
        out_specs=pl.BlockSpec((_BN, _D), lambda i: (i, 0)),
        out_shape=jax.ShapeDtypeStruct((_N, _D), jnp.float32),
    )(agg2, cnt2, r)


def kernel(graph, edge_index, edge_attr,
           g0, mu0, sigma0, root0, bias0,
           g1, mu1, sigma1, root1, bias1,
           g2, mu2, sigma2, root2, bias2):
    src = edge_index[0]
    dst = edge_index[1]
    src2 = src.reshape(_ER, _B)
    dst2 = dst.reshape(_ER, _B)
    ea02 = jnp.asarray(edge_attr[:, 0]).reshape(_ER, _B)
    ea12 = jnp.asarray(edge_attr[:, 1]).reshape(_ER, _B)
    zero = jnp.zeros((_N, _D), jnp.float32)
    zero16 = jnp.zeros((_N, 16), jnp.float32)

    mzivs, gcats, biases = [], [], []
    for (g, mu, sigma, root, bias) in ((g0, mu0, sigma0, root0, bias0),
                                       (g1, mu1, sigma1, root1, bias1),
                                       (g2, mu2, sigma2, root2, bias2)):
        mu_t = mu.T
        iv_t = 1.0 / (_EPS + sigma.T ** 2)
        packed = jnp.concatenate([mu_t, iv_t], axis=0)
        mzivs.append(jnp.stack([
            jnp.pad(packed[:, c * _KH:(c + 1) * _KH], ((0, 0), (0, _KP - _KH)))
            for c in range(_NC)]))
        gcats.append(jnp.concatenate(
            [g[:, _PERMS[0]], g[:, _PERMS[1]], root], axis=1))
        biases.append(bias.reshape(1, _D))

    ones16 = jnp.ones((_BC, 16), jnp.float32)
    cnt2 = _get_cnt_pass()(dst, zero16, ones16)

    outs = []
    h = graph
    for l in range(3):
        y, r = _node_transform(h, gcats[l], biases[l])
        agg2 = _get_edge_pass()(y, ea02, ea12, src2, dst2, mzivs[l], zero, cnt2)
        h = _combine(agg2, cnt2, r)
        outs.append(h)
    return tuple(outs)


# speedup vs baseline: 2.3319x; 1.1666x over previous
import functools

import jax
import jax.numpy as jnp
import numpy as np
from jax import lax
from jax.experimental import pallas as pl
from jax.experimental.pallas import tpu as pltpu
from jax.experimental.pallas import tpu_sc as plsc

_N = 10000
_E = 320000
_K = 10
_KH = 5
_D = 128
_DH = _KH * _D
_KP = 16
_EPS = 1e-15

_NC = 2
_NS = 16
_ET = _E // _NS
_B = 32
_SB = 5
_NSLOT = 4
_NSB = _ET // (_B * _SB)
_ER = _E // _B
_RPT = _ET // _B
_NPT = _N // _NS

_EW = _E // (_NC * _NS)
_BC = 80


def _half_perm(c):
    perm = []
    for g0 in range(20):
        k, s2 = divmod(g0, 4)
        base = (c * _KH + k) * _D + s2 * 32
        for i in range(16):
            perm.append(base + i)
            perm.append(base + 16 + i)
    return np.asarray(perm, dtype=np.int32)


_PERMS = (_half_perm(0), _half_perm(1))


_BN = 1000


def _y_body(h_ref, gcat_ref, bias_ref, y_ref, r_ref):
    hcat = jax.lax.dot_general(
        h_ref[...], gcat_ref[...], (((1,), (0,)), ((), ())),
        preferred_element_type=jnp.float32,
        precision=jax.lax.Precision.HIGHEST)
    y_ref[0] = hcat[:, :_DH].astype(jnp.bfloat16)
    y_ref[1] = hcat[:, _DH:2 * _DH].astype(jnp.bfloat16)
    r_ref[...] = hcat[:, 2 * _DH:] + bias_ref[...]


def _node_transform(h, gcat, bias2d):
    grid = _N // _BN
    return pl.pallas_call(
        _y_body,
        grid=(grid,),
        in_specs=[pl.BlockSpec((_BN, _D), lambda i: (i, 0)),
                  pl.BlockSpec((_D, 2 * _DH + _D), lambda i: (0, 0)),
                  pl.BlockSpec((1, _D), lambda i: (0, 0))],
        out_specs=[pl.BlockSpec((_NC, _BN, _DH), lambda i: (0, i, 0)),
                   pl.BlockSpec((_BN, _D), lambda i: (i, 0))],
        out_shape=[jax.ShapeDtypeStruct((_NC, _N, _DH), jnp.bfloat16),
                   jax.ShapeDtypeStruct((_N, _D), jnp.float32)],
    )(h, gcat, bias2d)


def _cnt_body(dst_hbm, zero_hbm, ones_hbm, out_hbm, dst_v, one_v, cnt_sh):
    cid = lax.axis_index("c")
    sid = lax.axis_index("s")
    wid = cid * _NS + sid

    pltpu.sync_copy(zero_hbm.at[pl.ds(sid * _NPT, _NPT)],
                    cnt_sh.at[pl.ds(sid * _NPT, _NPT)])
    pltpu.sync_copy(ones_hbm, one_v)
    plsc.subcore_barrier()

    base0 = wid * _EW

    def block_body(i, carry):
        pltpu.sync_copy(dst_hbm.at[pl.ds(base0 + i * _BC, _BC)], dst_v)
        pltpu.sync_copy(one_v, cnt_sh.at[dst_v], add=True)
        return carry
    lax.fori_loop(0, _EW // _BC, block_body, 0)

    plsc.subcore_barrier()
    pltpu.sync_copy(cnt_sh.at[pl.ds(sid * _NPT, _NPT)],
                    out_hbm.at[cid, pl.ds(sid * _NPT, _NPT)])


@functools.cache
def _get_cnt_pass():
    mesh = plsc.VectorSubcoreMesh(core_axis_name="c", subcore_axis_name="s",
                                  num_cores=_NC, num_subcores=_NS)
    return pl.kernel(
        _cnt_body,
        out_type=jax.ShapeDtypeStruct((_NC, _N, 16), jnp.float32),
        mesh=mesh,
        scratch_types=[
            pltpu.VMEM((_BC,), jnp.int32),
            pltpu.VMEM((_BC, 16), jnp.float32),
            pltpu.VMEM_SHARED((_N, 16), jnp.float32),
        ],
        compiler_params=pltpu.CompilerParams(use_tc_tiling_on_sc=False),
    )


def _edge_body(y_hbm, eidx_hbm, mziv_hbm, zero_hbm,
               cnt_hbm, out_hbm, eidx_v,
               rows0_v, rows1_v, rows2_v, rows3_v, msg0_v, msg1_v, mz_v,
               agg_sh, gsem0, gsem1, gsem2, gsem3, ssem0, ssem1):
    cid = lax.axis_index("c")
    sid = lax.axis_index("s")
    row0 = sid * _RPT

    pltpu.sync_copy(zero_hbm.at[pl.ds(sid * _NPT, _NPT)],
                    agg_sh.at[pl.ds(sid * _NPT, _NPT)])
    pltpu.sync_copy(mziv_hbm.at[cid], mz_v)
    plsc.subcore_barrier()

    mu0 = mz_v[0, pl.ds(0, 16)]
    mu1 = mz_v[1, pl.ds(0, 16)]
    iv0 = mz_v[2, pl.ds(0, 16)]
    iv1 = mz_v[3, pl.ds(0, 16)]

    rows_bufs = (rows0_v, rows1_v, rows2_v, rows3_v)
    gsems = (gsem0, gsem1, gsem2, gsem3)
    msg_bufs = (msg0_v, msg1_v)
    ssems = (ssem0, ssem1)

    def gather_start(j, slot):
        pltpu.async_copy(y_hbm.at[cid].at[eidx_v.at[j, 0]], rows_bufs[slot],
                         gsems[slot])

    def gather_wait(j, slot):
        pltpu.make_async_copy(y_hbm.at[cid].at[eidx_v.at[j, 0]],
                              rows_bufs[slot], gsems[slot]).wait()

    def scatter_wait(j, slot):
        pltpu.make_async_copy(msg_bufs[slot], agg_sh.at[eidx_v.at[j, 1]],
                              ssems[slot]).wait()

    def compute_block(j, slot):
        rows_v = rows_bufs[slot]
        msg_v = msg_bufs[j % 2]

        def group_one(gi, c2):
            b0 = gi * 16
            a0 = plsc.bitcast(eidx_v[j, 2, pl.ds(b0, 16)], jnp.float32)
            a1 = plsc.bitcast(eidx_v[j, 3, pl.ds(b0, 16)], jnp.float32)
            wvs = []
            for k in range(_KH):
                d0 = a0 - mu0[k]
                d1 = a1 - mu1[k]
                expo = -0.5 * (d0 * d0 * iv0[k] + d1 * d1 * iv1[k])
                wvs.append(jnp.exp(expo))
            for jj in range(16):
                b = b0 + jj
                accs = [None] * (_D // 32)
                for k in range(_KH):
                    wsp = lax.gather(
                        wvs[k], jnp.full((16, 1), jj, jnp.int32),
                        lax.GatherDimensionNumbers(
                            offset_dims=(), collapsed_slice_dims=(0,),
                            start_index_map=(0,)),
                        (1,), mode=lax.GatherScatterMode.PROMISE_IN_BOUNDS)
                    wkj = plsc.pack(wsp, wsp,
                                    format=plsc.PackFormat.INTERLEAVED)
                    for s2 in range(_D // 32):
                        t = wkj * rows_v[b, pl.ds(k * _D + s2 * 32, 32)]
                        accs[s2] = t if accs[s2] is None else accs[s2] + t
                for s2 in range(_D // 32):
                    lo, hi = plsc.unpack(
                        accs[s2], format=plsc.PackFormat.INTERLEAVED)
                    msg_v[b, pl.ds(2 * s2 * 16, 16)] = lo
                    msg_v[b, pl.ds((2 * s2 + 1) * 16, 16)] = hi
            return c2
        lax.fori_loop(0, _B // 16, group_one, 0)
        pltpu.async_copy(msg_v, agg_sh.at[eidx_v.at[j, 1]], ssems[j % 2],
                         add=True)

    def super_body(g, carry):
        r = row0 + g * _SB
        pltpu.sync_copy(eidx_hbm.at[pl.ds(r, _SB)], eidx_v)
        for j in range(_NSLOT - 1):
            gather_start(j, j % _NSLOT)
        for j in range(_SB):
            gather_wait(j, j % _NSLOT)
            if j >= 2:
                scatter_wait(j - 2, j % 2)
            compute_block(j, j % _NSLOT)
            if j + _NSLOT - 1 < _SB:
                gather_start(j + _NSLOT - 1, (j + _NSLOT - 1) % _NSLOT)
        scatter_wait(_SB - 2, (_SB - 2) % 2)
        scatter_wait(_SB - 1, (_SB - 1) % 2)
        return carry

    lax.fori_loop(0, _NSB, super_body, 0)

    plsc.subcore_barrier()
    pltpu.sync_copy(agg_sh.at[pl.ds(sid * _NPT, _NPT)],
                    out_hbm.at[cid, pl.ds(sid * _NPT, _NPT)])


@functools.cache
def _get_edge_pass():
    mesh = plsc.VectorSubcoreMesh(core_axis_name="c", subcore_axis_name="s",
                                  num_cores=_NC, num_subcores=_NS)
    return pl.kernel(
        _edge_body,
        out_type=jax.ShapeDtypeStruct((_NC, _N, _D), jnp.float32),
        mesh=mesh,
        scratch_types=[
            pltpu.VMEM((_SB, 4, _B), jnp.int32),
            pltpu.VMEM((_B, _DH), jnp.bfloat16),
            pltpu.VMEM((_B, _DH), jnp.bfloat16),
            pltpu.VMEM((_B, _DH), jnp.bfloat16),
            pltpu.VMEM((_B, _DH), jnp.bfloat16),
            pltpu.VMEM((_B, _D), jnp.float32),
            pltpu.VMEM((_B, _D), jnp.float32),
            pltpu.VMEM((4, 16), jnp.float32),
            pltpu.VMEM_SHARED((_N, _D), jnp.float32),
            pltpu.SemaphoreType.DMA,
            pltpu.SemaphoreType.DMA,
            pltpu.SemaphoreType.DMA,
            pltpu.SemaphoreType.DMA,
            pltpu.SemaphoreType.DMA,
            pltpu.SemaphoreType.DMA,
        ],
        compiler_params=pltpu.CompilerParams(use_tc_tiling_on_sc=False,
                                             needs_layout_passes=False),
    )


def _combine_body(agg_ref, cnt_ref, r_ref, h_ref):
    s = agg_ref[0] + agg_ref[1]
    cnt = cnt_ref[0, :, 0:1] + cnt_ref[1, :, 0:1]
    o = s / jnp.maximum(cnt, 1.0) + r_ref[...]
    h_ref[...] = jnp.where(o > 0, o, jnp.exp(o) - 1.0)


def _combine(agg2, cnt2, r):
    grid = _N // _BN
    return pl.pallas_call(
        _combine_body,
        grid=(grid,),
        in_specs=[pl.BlockSpec((_NC, _BN, _D), lambda i: (0, i, 0)),
                  pl.BlockSpec((_NC, _BN, 16), lambda i: (0, i, 0)),
                  pl.BlockSpec((_BN, _D), lambda i: (i, 0))],
        out_specs=pl.BlockSpec((_BN, _D), lambda i: (i, 0)),
        out_shape=jax.ShapeDtypeStruct((_N, _D), jnp.float32),
    )(agg2, cnt2, r)


def kernel(graph, edge_index, edge_attr,
           g0, mu0, sigma0, root0, bias0,
           g1, mu1, sigma1, root1, bias1,
           g2, mu2, sigma2, root2, bias2):
    src = edge_index[0]
    dst = edge_index[1]
    eidx = jnp.stack([
        src.reshape(_ER, _B),
        dst.reshape(_ER, _B),
        jax.lax.bitcast_convert_type(
            jnp.asarray(edge_attr[:, 0]), jnp.int32).reshape(_ER, _B),
        jax.lax.bitcast_convert_type(
            jnp.asarray(edge_attr[:, 1]), jnp.int32).reshape(_ER, _B),
    ], axis=1)
    zero = jnp.zeros((_N, _D), jnp.float32)
    zero16 = jnp.zeros((_N, 16), jnp.float32)

    mzivs, gcats, biases = [], [], []
    for (g, mu, sigma, root, bias) in ((g0, mu0, sigma0, root0, bias0),
                                       (g1, mu1, sigma1, root1, bias1),
                                       (g2, mu2, sigma2, root2, bias2)):
        mu_t = mu.T
        iv_t = 1.0 / (_EPS + sigma.T ** 2)
        packed = jnp.concatenate([mu_t, iv_t], axis=0)
        mzivs.append(jnp.stack([
            jnp.pad(packed[:, c * _KH:(c + 1) * _KH], ((0, 0), (0, _KP - _KH)))
            for c in range(_NC)]))
        gcats.append(jnp.concatenate(
            [g[:, _PERMS[0]], g[:, _PERMS[1]], root], axis=1))
        biases.append(bias.reshape(1, _D))

    ones16 = jnp.ones((_BC, 16), jnp.float32)
    cnt2 = _get_cnt_pass()(dst, zero16, ones16)

    outs = []
    h = graph
    for l in range(3):
        y, r = _node_transform(h, gcats[l], biases[l])
        agg2 = _get_edge_pass()(y, eidx, mzivs[l], zero, cnt2)
        h = _combine(agg2, cnt2, r)
        outs.append(h)
    return tuple(outs)
